# baseline (speedup 1.0000x reference)
import functools

import jax
import jax.numpy as jnp
from jax import lax
from jax.experimental import pallas as pl
from jax.experimental.pallas import tpu as pltpu
from jax.experimental.pallas import tpu_sc as plsc


def _sc_gather4(u_idx4, m_idx4, utab4, mtab4):
    NWi, NCH, CHUNK = u_idx4.shape
    W = utab4.shape[1]
    info = plsc.get_sparse_core_info()
    NC = info.num_cores
    mesh = plsc.VectorSubcoreMesh(core_axis_name="c", subcore_axis_name="s")

    @functools.partial(
        pl.kernel,
        mesh=mesh,
        out_type=[
            jax.ShapeDtypeStruct((NWi, NCH, CHUNK, W), jnp.float32),
            jax.ShapeDtypeStruct((NWi, NCH, CHUNK, W), jnp.float32),
        ],
        scratch_types=[
            pltpu.VMEM((NCH, CHUNK), jnp.int32),
            pltpu.VMEM((NCH, CHUNK), jnp.int32),
            pltpu.VMEM((CHUNK, W), jnp.float32),
            pltpu.VMEM((CHUNK, W), jnp.float32),
            pltpu.SemaphoreType.DMA,
        ],
    )
    def gather_kernel(u_idx_hbm, m_idx_hbm, utab_hbm, mtab_hbm,
                      u_out_hbm, m_out_hbm,
                      uidx_v, midx_v, urows_v, mrows_v, sem):
        wid = lax.axis_index("s") * NC + lax.axis_index("c")
        pltpu.sync_copy(u_idx_hbm.at[wid], uidx_v)
        pltpu.sync_copy(m_idx_hbm.at[wid], midx_v)
        for j in range(NCH):
            cu = pltpu.async_copy(utab_hbm.at[uidx_v.at[j]], urows_v, sem)
            cm = pltpu.async_copy(mtab_hbm.at[midx_v.at[j]], mrows_v, sem)
            cu.wait()
            cm.wait()
            pltpu.sync_copy(urows_v, u_out_hbm.at[wid].at[j])
            pltpu.sync_copy(mrows_v, m_out_hbm.at[wid].at[j])

    return gather_kernel(u_idx4, m_idx4, utab4, mtab4)


def _mlp_body(u_ref, m_ref, us_ref, ms_ref, w1u_ref, w1m_ref, b1_ref,
              w2_ref, b2_ref, w3_ref, b3_ref, out_ref):
    blk, W = u_ref.shape
    lane_grp = lax.broadcasted_iota(jnp.int32, (blk, W), 1) >> 5
    usel = (us_ref[...] & 3) + jnp.zeros((blk, W), jnp.int32)
    msel = (ms_ref[...] & 3) + jnp.zeros((blk, W), jnp.int32)
    u = jnp.where(lane_grp == usel, u_ref[...], 0.0)
    m = jnp.where(lane_grp == msel, m_ref[...], 0.0)
    x = jnp.dot(u, w1u_ref[...], preferred_element_type=jnp.float32)
    x = x + jnp.dot(m, w1m_ref[...], preferred_element_type=jnp.float32)
    h1 = jnp.maximum(x + b1_ref[...], 0.0)
    h2 = jnp.dot(h1, w2_ref[...], preferred_element_type=jnp.float32)
    h2 = jnp.maximum(h2 + b2_ref[...], 0.0)
    o = jnp.sum(h2 * w3_ref[...], axis=1) + b3_ref[0, 0]
    out_ref[...] = o


def _tc_mlp4(u128, m128, users, movies, W1, b1, W2, b2, W3, b3, blk=2048):
    B, W = u128.shape
    E = W1.shape[1] // 2
    H1 = W1.shape[0]
    H2 = W2.shape[0]
    w1u4 = jnp.tile(W1[:, :E].T, (W // E, 1))
    w1m4 = jnp.tile(W1[:, E:].T, (W // E, 1))
    w2t = W2.T
    b1r = b1.reshape(1, H1)
    b2r = b2.reshape(1, H2)
    w3r = W3.reshape(1, H2)
    b3r = b3.reshape(1, 1)
    us2 = users.reshape(B, 1)
    ms2 = movies.reshape(B, 1)

    grid = (B // blk,)
    full = lambda i: (0, 0)
    return pl.pallas_call(
        _mlp_body,
        grid=grid,
        in_specs=[
            pl.BlockSpec((blk, W), lambda i: (i, 0)),
            pl.BlockSpec((blk, W), lambda i: (i, 0)),
            pl.BlockSpec((blk, 1), lambda i: (i, 0)),
            pl.BlockSpec((blk, 1), lambda i: (i, 0)),
            pl.BlockSpec((W, H1), full),
            pl.BlockSpec((W, H1), full),
            pl.BlockSpec((1, H1), full),
            pl.BlockSpec((H1, H2), full),
            pl.BlockSpec((1, H2), full),
            pl.BlockSpec((1, H2), full),
            pl.BlockSpec((1, 1), full),
        ],
        out_specs=pl.BlockSpec((blk,), lambda i: (i,)),
        out_shape=jax.ShapeDtypeStruct((B,), jnp.float32),
    )(u128, m128, us2, ms2, w1u4, w1m4, b1r, w2t, b2r, w3r, b3r)


def kernel(users, movies, user_table, movie_table, W1, b1, W2, b2, W3, b3):
    B = users.shape[0]
    V, E = user_table.shape
    NW = 32
    CHUNK = 128
    NCH = B // (NW * CHUNK)
    W = 4 * E
    utab4 = user_table.reshape(V // 4, W)
    mtab4 = movie_table.reshape(movie_table.shape[0] // 4, W)
    u_idx4 = (users >> 2).reshape(NW, NCH, CHUNK)
    m_idx4 = (movies >> 2).reshape(NW, NCH, CHUNK)
    u128, m128 = _sc_gather4(u_idx4, m_idx4, utab4, mtab4)
    return _tc_mlp4(u128.reshape(B, W), m128.reshape(B, W),
                    users, movies, W1, b1, W2, b2, W3, b3)


# speedup vs baseline: 1.5045x; 1.5045x over previous
import functools

import jax
import jax.numpy as jnp
from jax import lax
from jax.experimental import pallas as pl
from jax.experimental.pallas import tpu as pltpu
from jax.experimental.pallas import tpu_sc as plsc


def _sc_gather_one(idx, table):
    B = idx.shape[0]
    E = table.shape[1]
    info = plsc.get_sparse_core_info()
    NC, NS = info.num_cores, info.num_subcores
    NW = NC * NS
    BPW = B // NW
    G = BPW // 16

    mesh = plsc.VectorSubcoreMesh(core_axis_name="c", subcore_axis_name="s")

    @functools.partial(
        pl.kernel,
        mesh=mesh,
        out_type=jax.ShapeDtypeStruct((B, E), jnp.float32),
        scratch_types=[
            pltpu.VMEM((BPW,), jnp.int32),
            pltpu.VMEM((BPW // 2, E), jnp.float32),
            pltpu.SemaphoreType.DMA,
        ],
    )
    def gather_kernel(idx_hbm, tab_hbm, out_hbm, idx_v, rows_v, sem):
        wid = lax.axis_index("s") * NC + lax.axis_index("c")
        base = wid * BPW
        half = BPW // 2
        pltpu.sync_copy(idx_hbm.at[pl.ds(base, BPW)], idx_v)

        def fire_group(h, g):
            vec = idx_v[pl.ds(h * half + g * 16, 16)]
            for j in range(16):
                pltpu.async_copy(tab_hbm.at[pl.ds(vec[j], 1)],
                                 rows_v.at[pl.ds(g * 16 + j, 1)], sem)

        def drain_group():
            for _ in range(16):
                pltpu.make_async_copy(tab_hbm.at[pl.ds(0, 1)],
                                      rows_v.at[pl.ds(0, 1)], sem).wait()

        for h in range(2):
            fire_group(h, 0)

            def body(g, _, h=h):
                fire_group(h, g)
                drain_group()
                return ()

            lax.fori_loop(1, G // 2, body, ())
            drain_group()
            pltpu.sync_copy(rows_v, out_hbm.at[pl.ds(base + h * half, half)])

    return gather_kernel(idx, table)


def _mlp_body(u_ref, m_ref, w1u_ref, w1m_ref, b1_ref, w2_ref, b2_ref,
              w3_ref, b3_ref, out_ref):
    x = jnp.dot(u_ref[...], w1u_ref[...], preferred_element_type=jnp.float32)
    x = x + jnp.dot(m_ref[...], w1m_ref[...],
                    preferred_element_type=jnp.float32)
    h1 = jnp.maximum(x + b1_ref[...], 0.0)
    h2 = jnp.dot(h1, w2_ref[...], preferred_element_type=jnp.float32)
    h2 = jnp.maximum(h2 + b2_ref[...], 0.0)
    o = jnp.sum(h2 * w3_ref[...], axis=1) + b3_ref[0, 0]
    out_ref[...] = o


def _tc_mlp(u, m, W1, b1, W2, b2, W3, b3, blk=4096):
    B, E = u.shape
    H1 = W1.shape[0]
    H2 = W2.shape[0]
    w1u = W1[:, :E].T
    w1m = W1[:, E:].T
    w2t = W2.T
    b1r = b1.reshape(1, H1)
    b2r = b2.reshape(1, H2)
    w3r = W3.reshape(1, H2)
    b3r = b3.reshape(1, 1)

    grid = (B // blk,)
    full = lambda i: (0, 0)
    return pl.pallas_call(
        _mlp_body,
        grid=grid,
        in_specs=[
            pl.BlockSpec((blk, E), lambda i: (i, 0)),
            pl.BlockSpec((blk, E), lambda i: (i, 0)),
            pl.BlockSpec((E, H1), full),
            pl.BlockSpec((E, H1), full),
            pl.BlockSpec((1, H1), full),
            pl.BlockSpec((H1, H2), full),
            pl.BlockSpec((1, H2), full),
            pl.BlockSpec((1, H2), full),
            pl.BlockSpec((1, 1), full),
        ],
        out_specs=pl.BlockSpec((blk,), lambda i: (i,)),
        out_shape=jax.ShapeDtypeStruct((B,), jnp.float32),
    )(u, m, w1u, w1m, b1r, w2t, b2r, w3r, b3r)


def kernel(users, movies, user_table, movie_table, W1, b1, W2, b2, W3, b3):
    m = _sc_gather_one(movies, movie_table)
    u = _sc_gather_one(users, user_table)
    return _tc_mlp(u, m, W1, b1, W2, b2, W3, b3)
